# Initial kernel scaffold; baseline (speedup 1.0000x reference)
import jax
import jax.numpy as jnp
from jax.experimental import pallas as pl


def kernel(x, b, k, W_qv, W_out, b_out):
    raise NotImplementedError("write your pallas kernel here")


# speedup vs baseline: 27.2928x; 27.2928x over previous
import jax
import jax.numpy as jnp
from jax.experimental import pallas as pl
from jax.experimental.pallas import tpu as pltpu

_B, _I, _DIM = 1, 2048, 1024
_H, _DH = 8, 64
_NB = 64
_INNER = _H * _DH
_T = 256


def _fused_body(x_ref, b_ref, k_ref, wqv_ref, wout_ref, bout_ref, out_ref):
    xb = x_ref[...]
    qv = jax.lax.dot_general(
        xb, wqv_ref[...], (((1,), (0,)), ((), ())),
        preferred_element_type=jnp.float32)

    scale = jnp.float32(_DH ** -0.5)
    k_mat = k_ref[...]

    b_full = b_ref[...]
    d_iota = jax.lax.broadcasted_iota(jnp.int32, (1, _DH, _DH), 1)
    e_iota = jax.lax.broadcasted_iota(jnp.int32, (1, _DH, _DH), 2)
    eye = (d_iota == e_iota).astype(jnp.float32)
    diag_b = jnp.sum(b_full * eye, axis=2)

    n_iota = jax.lax.broadcasted_iota(jnp.int32, (_T, _NB), 1)
    big = jnp.int32(_NB)

    outs = []
    for h in range(_H):
        q_h = qv[:, h * _DH:(h + 1) * _DH]
        v_h = qv[:, _INNER + h * _DH:_INNER + (h + 1) * _DH]
        sim = jax.lax.dot_general(
            q_h, k_mat, (((1,), (1,)), ((), ())),
            preferred_element_type=jnp.float32) * scale
        m1 = jnp.max(sim, axis=1, keepdims=True)
        i1 = jnp.min(jnp.where(sim == m1, n_iota, big), axis=1, keepdims=True)
        mask1 = n_iota == i1
        sim2 = jnp.where(mask1, -jnp.inf, sim)
        m2 = jnp.max(sim2, axis=1, keepdims=True)
        i2 = jnp.min(jnp.where(sim2 == m2, n_iota, big), axis=1, keepdims=True)
        mask2 = n_iota == i2
        e2 = jnp.exp(m2 - m1)
        denom = 1.0 + e2
        a1 = 1.0 / denom
        a2 = e2 / denom
        w = jnp.where(mask1, a1, 0.0) + jnp.where(mask2, a2, 0.0)
        eff = jax.lax.dot_general(
            w, diag_b, (((1,), (0,)), ((), ())),
            preferred_element_type=jnp.float32)
        outs.append(eff * v_h)

    acc = jnp.concatenate(outs, axis=1)
    res = jax.lax.dot_general(
        acc, wout_ref[...], (((1,), (0,)), ((), ())),
        preferred_element_type=jnp.float32)
    out_ref[...] = res + bout_ref[...]


def kernel(x, b, k, W_qv, W_out, b_out):
    x2 = x.reshape(_I, _DIM)
    bout2 = b_out.reshape(1, _DIM)
    grid = (_I // _T,)
    out = pl.pallas_call(
        _fused_body,
        grid=grid,
        in_specs=[
            pl.BlockSpec((_T, _DIM), lambda i: (i, 0)),
            pl.BlockSpec((_NB, _DH, _DH), lambda i: (0, 0, 0)),
            pl.BlockSpec((_NB, _DH), lambda i: (0, 0)),
            pl.BlockSpec((_DIM, 2 * _INNER), lambda i: (0, 0)),
            pl.BlockSpec((_INNER, _DIM), lambda i: (0, 0)),
            pl.BlockSpec((1, _DIM), lambda i: (0, 0)),
        ],
        out_specs=pl.BlockSpec((_T, _DIM), lambda i: (i, 0)),
        out_shape=jax.ShapeDtypeStruct((_I, _DIM), jnp.float32),
        compiler_params=pltpu.CompilerParams(
            dimension_semantics=("parallel",),
        ),
    )(x2, b, k, W_qv, W_out, bout2)
    return out.reshape(_B, _I, _DIM)


# speedup vs baseline: 32.2499x; 1.1816x over previous
import jax
import jax.numpy as jnp
from jax.experimental import pallas as pl
from jax.experimental.pallas import tpu as pltpu

_B, _I, _DIM = 1, 2048, 1024
_H, _DH = 8, 64
_NB = 64
_INNER = _H * _DH
_T = 256


def _fused_body(x_ref, wq_ref, wv_ref, k_ref, diag_ref, wo_ref, bout_ref,
                out_ref):
    xb = x_ref[...]
    q = jax.lax.dot_general(
        xb, wq_ref[...], (((1,), (0,)), ((), ())),
        preferred_element_type=jnp.float32)
    v = jax.lax.dot_general(
        xb.astype(jnp.bfloat16), wv_ref[...], (((1,), (0,)), ((), ())),
        preferred_element_type=jnp.float32)

    scale = jnp.float32(_DH ** -0.5)
    k_mat = k_ref[...]
    diag_b = diag_ref[...]

    f_iota = jax.lax.broadcasted_iota(
        jnp.int32, (_T, _NB), 1).astype(jnp.float32)
    big = jnp.float32(_NB)

    outs = []
    for h in range(_H):
        q_h = q[:, h * _DH:(h + 1) * _DH]
        v_h = v[:, h * _DH:(h + 1) * _DH]
        sim = jax.lax.dot_general(
            q_h, k_mat, (((1,), (1,)), ((), ())),
            preferred_element_type=jnp.float32) * scale
        m1 = jnp.max(sim, axis=1, keepdims=True)
        i1 = jnp.min(jnp.where(sim == m1, f_iota, big), axis=1, keepdims=True)
        mask1 = f_iota == i1
        sim2 = jnp.where(mask1, -jnp.inf, sim)
        m2 = jnp.max(sim2, axis=1, keepdims=True)
        i2 = jnp.min(jnp.where(sim2 == m2, f_iota, big), axis=1, keepdims=True)
        mask2 = f_iota == i2
        e2 = jnp.exp(m2 - m1)
        denom = 1.0 + e2
        a1 = 1.0 / denom
        a2 = e2 / denom
        w = jnp.where(mask1, a1, 0.0) + jnp.where(mask2, a2, 0.0)
        eff = jax.lax.dot_general(
            w, diag_b, (((1,), (0,)), ((), ())),
            preferred_element_type=jnp.float32)
        outs.append((eff * v_h).astype(jnp.bfloat16))

    acc = jnp.concatenate(outs, axis=1)
    res = jax.lax.dot_general(
        acc, wo_ref[...], (((1,), (0,)), ((), ())),
        preferred_element_type=jnp.float32)
    out_ref[...] = res + bout_ref[...]


def kernel(x, b, k, W_qv, W_out, b_out):
    x2 = x.reshape(_I, _DIM)
    diag_b = jnp.diagonal(b, axis1=1, axis2=2)
    W_q = W_qv[:, :_INNER]
    W_v = W_qv[:, _INNER:].astype(jnp.bfloat16)
    W_o = W_out.astype(jnp.bfloat16)
    bout2 = b_out.reshape(1, _DIM)
    grid = (_I // _T,)
    out = pl.pallas_call(
        _fused_body,
        grid=grid,
        in_specs=[
            pl.BlockSpec((_T, _DIM), lambda i: (i, 0)),
            pl.BlockSpec((_DIM, _INNER), lambda i: (0, 0)),
            pl.BlockSpec((_DIM, _INNER), lambda i: (0, 0)),
            pl.BlockSpec((_NB, _DH), lambda i: (0, 0)),
            pl.BlockSpec((_NB, _DH), lambda i: (0, 0)),
            pl.BlockSpec((_INNER, _DIM), lambda i: (0, 0)),
            pl.BlockSpec((1, _DIM), lambda i: (0, 0)),
        ],
        out_specs=pl.BlockSpec((_T, _DIM), lambda i: (i, 0)),
        out_shape=jax.ShapeDtypeStruct((_I, _DIM), jnp.float32),
        compiler_params=pltpu.CompilerParams(
            dimension_semantics=("parallel",),
        ),
    )(x2, W_q, W_v, k, diag_b, W_o, bout2)
    return out.reshape(_B, _I, _DIM)


# speedup vs baseline: 35.2866x; 1.0942x over previous
import jax
import jax.numpy as jnp
from jax.experimental import pallas as pl
from jax.experimental.pallas import tpu as pltpu

_B, _I, _DIM = 1, 2048, 1024
_H, _DH = 8, 64
_NB = 64
_INNER = _H * _DH
_T = 256


def _fused_body(x_ref, wq_ref, wv_ref, k_ref, diag_ref, wo_ref, bout_ref,
                out_ref):
    xb = x_ref[...]
    q = jax.lax.dot_general(
        xb, wq_ref[...], (((1,), (0,)), ((), ())),
        preferred_element_type=jnp.float32)
    v = jax.lax.dot_general(
        xb.astype(jnp.bfloat16), wv_ref[...], (((1,), (0,)), ((), ())),
        preferred_element_type=jnp.float32)

    scale = jnp.float32(_DH ** -0.5)
    k_mat = k_ref[...]
    diag_b = diag_ref[...]

    outs = []
    for h in range(_H):
        q_h = q[:, h * _DH:(h + 1) * _DH]
        v_h = v[:, h * _DH:(h + 1) * _DH]
        sim = jax.lax.dot_general(
            q_h, k_mat, (((1,), (1,)), ((), ())),
            preferred_element_type=jnp.float32) * scale
        m1 = jnp.max(sim, axis=1, keepdims=True)
        mask1 = sim == m1
        sim2 = jnp.where(mask1, -jnp.inf, sim)
        m2 = jnp.max(sim2, axis=1, keepdims=True)
        mask2 = sim2 == m2
        e2 = jnp.exp(m2 - m1)
        denom = 1.0 + e2
        a1 = 1.0 / denom
        a2 = e2 / denom
        w = jnp.where(mask1, a1, 0.0) + jnp.where(mask2, a2, 0.0)
        eff = jax.lax.dot_general(
            w, diag_b, (((1,), (0,)), ((), ())),
            preferred_element_type=jnp.float32)
        outs.append((eff * v_h).astype(jnp.bfloat16))

    acc = jnp.concatenate(outs, axis=1)
    res = jax.lax.dot_general(
        acc, wo_ref[...], (((1,), (0,)), ((), ())),
        preferred_element_type=jnp.float32)
    out_ref[...] = res + bout_ref[...]


def kernel(x, b, k, W_qv, W_out, b_out):
    x2 = x.reshape(_I, _DIM)
    diag_b = jnp.diagonal(b, axis1=1, axis2=2)
    W_q = W_qv[:, :_INNER]
    W_v = W_qv[:, _INNER:].astype(jnp.bfloat16)
    W_o = W_out.astype(jnp.bfloat16)
    bout2 = b_out.reshape(1, _DIM)
    grid = (_I // _T,)
    out = pl.pallas_call(
        _fused_body,
        grid=grid,
        in_specs=[
            pl.BlockSpec((_T, _DIM), lambda i: (i, 0)),
            pl.BlockSpec((_DIM, _INNER), lambda i: (0, 0)),
            pl.BlockSpec((_DIM, _INNER), lambda i: (0, 0)),
            pl.BlockSpec((_NB, _DH), lambda i: (0, 0)),
            pl.BlockSpec((_NB, _DH), lambda i: (0, 0)),
            pl.BlockSpec((_INNER, _DIM), lambda i: (0, 0)),
            pl.BlockSpec((1, _DIM), lambda i: (0, 0)),
        ],
        out_specs=pl.BlockSpec((_T, _DIM), lambda i: (i, 0)),
        out_shape=jax.ShapeDtypeStruct((_I, _DIM), jnp.float32),
        compiler_params=pltpu.CompilerParams(
            dimension_semantics=("parallel",),
        ),
    )(x2, W_q, W_v, k, diag_b, W_o, bout2)
    return out.reshape(_B, _I, _DIM)


# speedup vs baseline: 48.3928x; 1.3714x over previous
import jax
import jax.numpy as jnp
from jax.experimental import pallas as pl
from jax.experimental.pallas import tpu as pltpu

_B, _I, _DIM = 1, 2048, 1024
_H, _DH = 8, 64
_NB = 64
_INNER = _H * _DH
_T = 256


def _fused_body(x_ref, wqv_ref, b_ref, k_ref, wo_ref, bout_ref, out_ref,
                wv_bf_ref, wo_bf_ref, diag_ref):
    @pl.when(pl.program_id(0) == 0)
    def _init():
        wv_bf_ref[...] = wqv_ref[:, _INNER:].astype(jnp.bfloat16)
        wo_bf_ref[...] = wo_ref[...].astype(jnp.bfloat16)
        b_full = b_ref[...]
        d_iota = jax.lax.broadcasted_iota(jnp.int32, (1, _DH, _DH), 1)
        e_iota = jax.lax.broadcasted_iota(jnp.int32, (1, _DH, _DH), 2)
        eye = (d_iota == e_iota).astype(jnp.float32)
        diag_ref[...] = jnp.sum(b_full * eye, axis=2)

    xb = x_ref[...]
    q = jax.lax.dot_general(
        xb, wqv_ref[:, :_INNER], (((1,), (0,)), ((), ())),
        preferred_element_type=jnp.float32)
    v = jax.lax.dot_general(
        xb.astype(jnp.bfloat16), wv_bf_ref[...], (((1,), (0,)), ((), ())),
        preferred_element_type=jnp.float32)

    scale = jnp.float32(_DH ** -0.5)
    k_mat = k_ref[...]
    diag_b = diag_ref[...]

    outs = []
    for h in range(_H):
        q_h = q[:, h * _DH:(h + 1) * _DH]
        v_h = v[:, h * _DH:(h + 1) * _DH]
        sim = jax.lax.dot_general(
            q_h, k_mat, (((1,), (1,)), ((), ())),
            preferred_element_type=jnp.float32) * scale
        m1 = jnp.max(sim, axis=1, keepdims=True)
        mask1 = sim == m1
        sim2 = jnp.where(mask1, -jnp.inf, sim)
        m2 = jnp.max(sim2, axis=1, keepdims=True)
        mask2 = sim2 == m2
        e2 = jnp.exp(m2 - m1)
        denom = 1.0 + e2
        a1 = 1.0 / denom
        a2 = e2 / denom
        w = jnp.where(mask1, a1, 0.0) + jnp.where(mask2, a2, 0.0)
        eff = jax.lax.dot_general(
            w, diag_b, (((1,), (0,)), ((), ())),
            preferred_element_type=jnp.float32)
        outs.append((eff * v_h).astype(jnp.bfloat16))

    acc = jnp.concatenate(outs, axis=1)
    res = jax.lax.dot_general(
        acc, wo_bf_ref[...], (((1,), (0,)), ((), ())),
        preferred_element_type=jnp.float32)
    out_ref[...] = res + bout_ref[...]


def kernel(x, b, k, W_qv, W_out, b_out):
    x2 = x.reshape(_I, _DIM)
    bout2 = b_out.reshape(1, _DIM)
    grid = (_I // _T,)
    out = pl.pallas_call(
        _fused_body,
        grid=grid,
        in_specs=[
            pl.BlockSpec((_T, _DIM), lambda i: (i, 0)),
            pl.BlockSpec((_DIM, 2 * _INNER), lambda i: (0, 0)),
            pl.BlockSpec((_NB, _DH, _DH), lambda i: (0, 0, 0)),
            pl.BlockSpec((_NB, _DH), lambda i: (0, 0)),
            pl.BlockSpec((_INNER, _DIM), lambda i: (0, 0)),
            pl.BlockSpec((1, _DIM), lambda i: (0, 0)),
        ],
        out_specs=pl.BlockSpec((_T, _DIM), lambda i: (i, 0)),
        out_shape=jax.ShapeDtypeStruct((_I, _DIM), jnp.float32),
        scratch_shapes=[
            pltpu.VMEM((_DIM, _INNER), jnp.bfloat16),
            pltpu.VMEM((_INNER, _DIM), jnp.bfloat16),
            pltpu.VMEM((_NB, _DH), jnp.float32),
        ],
        compiler_params=pltpu.CompilerParams(
            dimension_semantics=("arbitrary",),
        ),
    )(x2, W_qv, b, k, W_out, bout2)
    return out.reshape(_B, _I, _DIM)


# speedup vs baseline: 51.3945x; 1.0620x over previous
import jax
import jax.numpy as jnp
from jax.experimental import pallas as pl
from jax.experimental.pallas import tpu as pltpu

_B, _I, _DIM = 1, 2048, 1024
_H, _DH = 8, 64
_NB = 64
_INNER = _H * _DH
_T = 512


def _fused_body(x_ref, wqv_ref, b_ref, k_ref, wo_ref, bout_ref, out_ref,
                wv_bf_ref, wo_bf_ref, diag_ref):
    @pl.when(pl.program_id(0) == 0)
    def _init():
        wv_bf_ref[...] = wqv_ref[:, _INNER:].astype(jnp.bfloat16)
        wo_bf_ref[...] = wo_ref[...].astype(jnp.bfloat16)
        b_full = b_ref[...]
        d_iota = jax.lax.broadcasted_iota(jnp.int32, (1, _DH, _DH), 1)
        e_iota = jax.lax.broadcasted_iota(jnp.int32, (1, _DH, _DH), 2)
        eye = (d_iota == e_iota).astype(jnp.float32)
        diag_ref[...] = jnp.sum(b_full * eye, axis=2)

    xb = x_ref[...]
    q = jax.lax.dot_general(
        xb, wqv_ref[:, :_INNER], (((1,), (0,)), ((), ())),
        preferred_element_type=jnp.float32)
    v = jax.lax.dot_general(
        xb.astype(jnp.bfloat16), wv_bf_ref[...], (((1,), (0,)), ((), ())),
        preferred_element_type=jnp.float32)

    scale = jnp.float32(_DH ** -0.5)
    k_mat = k_ref[...]
    diag_b = diag_ref[...]

    outs = []
    for h in range(_H):
        q_h = q[:, h * _DH:(h + 1) * _DH]
        v_h = v[:, h * _DH:(h + 1) * _DH]
        sim = jax.lax.dot_general(
            q_h, k_mat, (((1,), (1,)), ((), ())),
            preferred_element_type=jnp.float32) * scale
        m1 = jnp.max(sim, axis=1, keepdims=True)
        mask1 = sim == m1
        sim2 = jnp.where(mask1, -jnp.inf, sim)
        m2 = jnp.max(sim2, axis=1, keepdims=True)
        mask2 = sim2 == m2
        e2 = jnp.exp(m2 - m1)
        denom = 1.0 + e2
        a1 = 1.0 / denom
        a2 = e2 / denom
        w = jnp.where(mask1, a1, 0.0) + jnp.where(mask2, a2, 0.0)
        eff = jax.lax.dot_general(
            w, diag_b, (((1,), (0,)), ((), ())),
            preferred_element_type=jnp.float32)
        outs.append((eff * v_h).astype(jnp.bfloat16))

    acc = jnp.concatenate(outs, axis=1)
    res = jax.lax.dot_general(
        acc, wo_bf_ref[...], (((1,), (0,)), ((), ())),
        preferred_element_type=jnp.float32)
    out_ref[...] = res + bout_ref[...]


def kernel(x, b, k, W_qv, W_out, b_out):
    x2 = x.reshape(_I, _DIM)
    bout2 = b_out.reshape(1, _DIM)
    grid = (_I // _T,)
    out = pl.pallas_call(
        _fused_body,
        grid=grid,
        in_specs=[
            pl.BlockSpec((_T, _DIM), lambda i: (i, 0)),
            pl.BlockSpec((_DIM, 2 * _INNER), lambda i: (0, 0)),
            pl.BlockSpec((_NB, _DH, _DH), lambda i: (0, 0, 0)),
            pl.BlockSpec((_NB, _DH), lambda i: (0, 0)),
            pl.BlockSpec((_INNER, _DIM), lambda i: (0, 0)),
            pl.BlockSpec((1, _DIM), lambda i: (0, 0)),
        ],
        out_specs=pl.BlockSpec((_T, _DIM), lambda i: (i, 0)),
        out_shape=jax.ShapeDtypeStruct((_I, _DIM), jnp.float32),
        scratch_shapes=[
            pltpu.VMEM((_DIM, _INNER), jnp.bfloat16),
            pltpu.VMEM((_INNER, _DIM), jnp.bfloat16),
            pltpu.VMEM((_NB, _DH), jnp.float32),
        ],
        compiler_params=pltpu.CompilerParams(
            dimension_semantics=("arbitrary",),
        ),
    )(x2, W_qv, b, k, W_out, bout2)
    return out.reshape(_B, _I, _DIM)
